# baseline (speedup 1.0000x reference)
import functools

import jax
import jax.numpy as jnp
from jax import lax
from jax.experimental import pallas as pl
from jax.experimental.pallas import tpu as pltpu
from jax.experimental.pallas import tpu_sc as plsc

_NC = 2
_NS = 16
_NW = _NC * _NS
_NI = 1000
_NP = 16
_NT = 4096
_TT = _NT // 128


def _sc_body(x4, c2, out3, buf, cvec, outbuf):
    wid = lax.axis_index("s") * _NC + lax.axis_index("c")

    def per_item(k, carry):
        i = wid + _NW * k

        @pl.when(i < _NI)
        def _():
            pltpu.sync_copy(x4.at[i], buf)
            pltpu.sync_copy(c2.at[pl.ds(2 * i, 2)], cvec)
            cps = [cvec[h, pl.ds(16 * g, 16)]
                   for h in range(2) for g in range(8)]

            def per_tt(tt, carry2):
                for v2 in range(8):
                    acc = jnp.zeros((16,), jnp.float32)
                    for p in range(_NP):
                        acc = acc + buf[p, tt, pl.ds(16 * v2, 16)] * cps[p]
                    outbuf[tt, pl.ds(16 * v2, 16)] = acc
                return carry2

            lax.fori_loop(0, _TT, per_tt, 0)
            pltpu.sync_copy(outbuf, out3.at[i])

        return carry

    lax.fori_loop(0, (_NI + _NW - 1) // _NW, per_item, 0)


def kernel(x, coef):
    num_trips, num_items, num_params = x.shape
    xt = jnp.transpose(x, (1, 2, 0))
    x4 = xt.reshape(num_items, num_params, _TT, 128)
    c2 = jnp.repeat(coef.reshape(num_items, num_params), 16,
                    axis=-1).reshape(2 * num_items, 128)

    mesh = plsc.VectorSubcoreMesh(core_axis_name="c", subcore_axis_name="s")
    run = functools.partial(
        pl.kernel,
        mesh=mesh,
        out_type=jax.ShapeDtypeStruct((num_items, _TT, 128), jnp.float32),
        scratch_types=[
            pltpu.VMEM((num_params, _TT, 128), jnp.float32),
            pltpu.VMEM((2, 128), jnp.float32),
            pltpu.VMEM((_TT, 128), jnp.float32),
        ],
    )(_sc_body)
    out3 = run(x4, c2)
    return out3.reshape(num_items, num_trips).T


# speedup vs baseline: 4.7087x; 4.7087x over previous
import jax
import jax.numpy as jnp
from jax.experimental import pallas as pl

_IB = 64


def _body(x_ref, c_ref, o_ref):
    o_ref[...] = jnp.sum(x_ref[...] * c_ref[...][:, :, None], axis=1)


def kernel(x, coef):
    num_trips, num_items, num_params = x.shape
    xt = jnp.transpose(x, (1, 2, 0))
    outT = pl.pallas_call(
        _body,
        grid=(pl.cdiv(num_items, _IB),),
        in_specs=[
            pl.BlockSpec((_IB, num_params, num_trips), lambda i: (i, 0, 0)),
            pl.BlockSpec((_IB, num_params), lambda i: (i, 0)),
        ],
        out_specs=pl.BlockSpec((_IB, num_trips), lambda i: (i, 0)),
        out_shape=jax.ShapeDtypeStruct((num_items, num_trips), jnp.float32),
    )(xt, coef)
    return outT.T
